# initial kernel scaffold (unmeasured)
import jax
import jax.numpy as jnp
from jax import lax
from jax.experimental import pallas as pl
from jax.experimental.pallas import tpu as pltpu

N_DEV = 16
R_HOPS = 8
L_HOPS = 7
AMAX_ROUNDS = 4


def kernel(x, w_mat):
    m_per, k = x.shape
    n_per = w_mat.shape[1]

    def body(x_ref, w_ref, out_ref, bufR, bufL, amax_cur, amax_recv,
             sendR, recvR, sendL, recvL, asend, arecv):
        me = lax.axis_index("i")
        right = lax.rem(me + 1, N_DEV)
        left = lax.rem(me + N_DEV - 1, N_DEV)

        def gemm_block(xc):
            y = lax.dot_general(
                xc, w_ref[...], (((1,), (0,)), ((), ())),
                preferred_element_type=jnp.float32,
                precision=lax.Precision.HIGHEST,
            )
            return jnp.maximum(y, 0.0)

        out_ref[pl.ds(me * m_per, m_per), :] = gemm_block(x_ref[...])

        for s in range(R_HOPS):
            rdma_r = pltpu.make_async_remote_copy(
                src_ref=(x_ref if s == 0 else bufR.at[s - 1]),
                dst_ref=bufR.at[s],
                send_sem=sendR.at[s], recv_sem=recvR.at[s],
                device_id=(right,), device_id_type=pl.DeviceIdType.MESH,
            )
            rdma_r.start()
            rdma_l = None
            if s < L_HOPS:
                rdma_l = pltpu.make_async_remote_copy(
                    src_ref=(x_ref if s == 0 else bufL.at[s - 1]),
                    dst_ref=bufL.at[s],
                    send_sem=sendL.at[s], recv_sem=recvL.at[s],
                    device_id=(left,), device_id_type=pl.DeviceIdType.MESH,
                )
                rdma_l.start()
            rdma_r.wait()
            if rdma_l is not None:
                rdma_l.wait()

            origin_r = lax.rem(me + N_DEV - (s + 1), N_DEV)
            out_ref[pl.ds(origin_r * m_per, m_per), :] = gemm_block(
                bufR[s, :, :])
            if s < L_HOPS:
                origin_l = lax.rem(me + s + 1, N_DEV)
                out_ref[pl.ds(origin_l * m_per, m_per), :] = gemm_block(
                    bufL[s, :, :])

        amax_cur[...] = jnp.full((8, 128), jnp.max(out_ref[...]), jnp.float32)
        for r in range(AMAX_ROUNDS):
            partner = jnp.bitwise_xor(me, 1 << r)
            rd = pltpu.make_async_remote_copy(
                src_ref=amax_cur, dst_ref=amax_recv.at[r],
                send_sem=asend.at[r], recv_sem=arecv.at[r],
                device_id=(partner,), device_id_type=pl.DeviceIdType.MESH,
            )
            rd.start()
            rd.wait()
            amax_cur[...] = jnp.maximum(amax_cur[...], amax_recv[r, :, :])

        gmax = amax_cur[0, 0]
        scale = gmax / 448.0
        q = (out_ref[...] / scale).astype(jnp.float8_e4m3fn)
        out_ref[...] = q.astype(jnp.float32) * scale

    return pl.pallas_call(
        body,
        out_shape=jax.ShapeDtypeStruct((N_DEV * m_per, n_per), jnp.float32),
        in_specs=[
            pl.BlockSpec(memory_space=pltpu.VMEM),
            pl.BlockSpec(memory_space=pltpu.VMEM),
        ],
        out_specs=pl.BlockSpec(memory_space=pltpu.VMEM),
        scratch_shapes=[
            pltpu.VMEM((R_HOPS, m_per, k), jnp.float32),
            pltpu.VMEM((L_HOPS, m_per, k), jnp.float32),
            pltpu.VMEM((8, 128), jnp.float32),
            pltpu.VMEM((AMAX_ROUNDS, 8, 128), jnp.float32),
            pltpu.SemaphoreType.DMA((R_HOPS,)),
            pltpu.SemaphoreType.DMA((R_HOPS,)),
            pltpu.SemaphoreType.DMA((L_HOPS,)),
            pltpu.SemaphoreType.DMA((L_HOPS,)),
            pltpu.SemaphoreType.DMA((AMAX_ROUNDS,)),
            pltpu.SemaphoreType.DMA((AMAX_ROUNDS,)),
        ],
    )(x, w_mat)


# baseline (device time: 473436 ns/iter reference)
import jax
import jax.numpy as jnp
from jax import lax
from jax.experimental import pallas as pl
from jax.experimental.pallas import tpu as pltpu

N_DEV = 16
R_HOPS = 8
L_HOPS = 7
DEPTH = 4
AMAX_ROUNDS = 4


def kernel(x, w_mat):
    m_per, k = x.shape
    n_per = w_mat.shape[1]

    def body(x_ref, w_ref, out_ref, bufR, bufL, amax_cur, amax_recv,
             sendR, recvR, sendL, recvL, asend, arecv, creditR, creditL):
        me = lax.axis_index("i")
        right = lax.rem(me + 1, N_DEV)
        left = lax.rem(me + N_DEV - 1, N_DEV)

        def gemm_block(xc):
            y = lax.dot_general(
                xc, w_ref[...], (((1,), (0,)), ((), ())),
                preferred_element_type=jnp.float32,
                precision=lax.Precision.HIGHEST,
            )
            return jnp.maximum(y, 0.0)

        out_ref[pl.ds(me * m_per, m_per), :] = gemm_block(x_ref[...])

        for s in range(R_HOPS):
            if s >= DEPTH:
                pl.semaphore_wait(creditR, 1)
            rdma_r = pltpu.make_async_remote_copy(
                src_ref=(x_ref if s == 0 else bufR.at[(s - 1) % DEPTH]),
                dst_ref=bufR.at[s % DEPTH],
                send_sem=sendR.at[s], recv_sem=recvR.at[s],
                device_id=(right,), device_id_type=pl.DeviceIdType.MESH,
            )
            rdma_r.start()
            rdma_l = None
            if s < L_HOPS:
                if s >= DEPTH:
                    pl.semaphore_wait(creditL, 1)
                rdma_l = pltpu.make_async_remote_copy(
                    src_ref=(x_ref if s == 0 else bufL.at[(s - 1) % DEPTH]),
                    dst_ref=bufL.at[s % DEPTH],
                    send_sem=sendL.at[s], recv_sem=recvL.at[s],
                    device_id=(left,), device_id_type=pl.DeviceIdType.MESH,
                )
                rdma_l.start()
            rdma_r.wait()
            if rdma_l is not None:
                rdma_l.wait()

            origin_r = lax.rem(me + N_DEV - (s + 1), N_DEV)
            out_ref[pl.ds(origin_r * m_per, m_per), :] = gemm_block(
                bufR[s % DEPTH, :, :])
            if s < L_HOPS:
                origin_l = lax.rem(me + s + 1, N_DEV)
                out_ref[pl.ds(origin_l * m_per, m_per), :] = gemm_block(
                    bufL[s % DEPTH, :, :])

            if 1 <= s <= R_HOPS - DEPTH:
                pl.semaphore_signal(
                    creditR, inc=1, device_id=(left,),
                    device_id_type=pl.DeviceIdType.MESH)
            if 1 <= s <= L_HOPS - DEPTH:
                pl.semaphore_signal(
                    creditL, inc=1, device_id=(right,),
                    device_id_type=pl.DeviceIdType.MESH)

        amax_cur[...] = jnp.full((8, 128), jnp.max(out_ref[...]), jnp.float32)
        for r in range(AMAX_ROUNDS):
            partner = jnp.bitwise_xor(me, 1 << r)
            rd = pltpu.make_async_remote_copy(
                src_ref=amax_cur, dst_ref=amax_recv.at[r],
                send_sem=asend.at[r], recv_sem=arecv.at[r],
                device_id=(partner,), device_id_type=pl.DeviceIdType.MESH,
            )
            rd.start()
            rd.wait()
            amax_cur[...] = jnp.maximum(amax_cur[...], amax_recv[r, :, :])

        gmax = amax_cur[0, 0]
        scale = gmax / 448.0
        q = (out_ref[...] / scale).astype(jnp.float8_e4m3fn)
        out_ref[...] = q.astype(jnp.float32) * scale

    return pl.pallas_call(
        body,
        out_shape=jax.ShapeDtypeStruct((N_DEV * m_per, n_per), jnp.float32),
        in_specs=[
            pl.BlockSpec(memory_space=pltpu.VMEM),
            pl.BlockSpec(memory_space=pltpu.VMEM),
        ],
        out_specs=pl.BlockSpec(memory_space=pltpu.VMEM),
        scratch_shapes=[
            pltpu.VMEM((DEPTH, m_per, k), jnp.float32),
            pltpu.VMEM((DEPTH, m_per, k), jnp.float32),
            pltpu.VMEM((8, 128), jnp.float32),
            pltpu.VMEM((AMAX_ROUNDS, 8, 128), jnp.float32),
            pltpu.SemaphoreType.DMA((R_HOPS,)),
            pltpu.SemaphoreType.DMA((R_HOPS,)),
            pltpu.SemaphoreType.DMA((L_HOPS,)),
            pltpu.SemaphoreType.DMA((L_HOPS,)),
            pltpu.SemaphoreType.DMA((AMAX_ROUNDS,)),
            pltpu.SemaphoreType.DMA((AMAX_ROUNDS,)),
            pltpu.SemaphoreType.REGULAR,
            pltpu.SemaphoreType.REGULAR,
        ],
        compiler_params=pltpu.CompilerParams(
            vmem_limit_bytes=60 * 1024 * 1024,
        ),
    )(x, w_mat)


# device time: 410013 ns/iter; 1.1547x vs baseline; 1.1547x over previous
import jax
import jax.numpy as jnp
from jax import lax
from jax.experimental import pallas as pl
from jax.experimental.pallas import tpu as pltpu

N_DEV = 16
R_HOPS = 8
L_HOPS = 7
DEPTH = 4
N_PEERS = N_DEV - 1


def kernel(x, w_mat):
    m_per, k = x.shape
    n_per = w_mat.shape[1]

    def body(x_ref, w_ref, out_ref, bufR, bufL, amax_cur, amax_all,
             sendR, recvR, sendL, recvL, bsend, brecv, creditR, creditL):
        me = lax.axis_index("i")
        right = lax.rem(me + 1, N_DEV)
        left = lax.rem(me + N_DEV - 1, N_DEV)

        def gemm_block(xc):
            y = lax.dot_general(
                xc, w_ref[...], (((1,), (0,)), ((), ())),
                preferred_element_type=jnp.float32,
                precision=lax.Precision.HIGHEST,
            )
            return jnp.maximum(y, 0.0)

        def start_r(s):
            rd = pltpu.make_async_remote_copy(
                src_ref=(x_ref if s == 0 else bufR.at[(s - 1) % DEPTH]),
                dst_ref=bufR.at[s % DEPTH],
                send_sem=sendR.at[s], recv_sem=recvR.at[s],
                device_id=(right,), device_id_type=pl.DeviceIdType.MESH,
            )
            rd.start()
            return rd

        def start_l(s):
            rd = pltpu.make_async_remote_copy(
                src_ref=(x_ref if s == 0 else bufL.at[(s - 1) % DEPTH]),
                dst_ref=bufL.at[s % DEPTH],
                send_sem=sendL.at[s], recv_sem=recvL.at[s],
                device_id=(left,), device_id_type=pl.DeviceIdType.MESH,
            )
            rd.start()
            return rd

        rd_r = [None] * R_HOPS
        rd_l = [None] * L_HOPS
        rd_r[0] = start_r(0)
        rd_l[0] = start_l(0)

        y = gemm_block(x_ref[...])
        local_max = jnp.max(y)
        out_ref[pl.ds(me * m_per, m_per), :] = y

        for s in range(R_HOPS):
            rd_r[s].wait_recv()
            if s < L_HOPS:
                rd_l[s].wait_recv()

            if s + 1 < R_HOPS:
                if s + 1 >= DEPTH:
                    pl.semaphore_wait(creditR, 1)
                rd_r[s + 1] = start_r(s + 1)
            if s + 1 < L_HOPS:
                if s + 1 >= DEPTH:
                    pl.semaphore_wait(creditL, 1)
                rd_l[s + 1] = start_l(s + 1)

            origin_r = lax.rem(me + N_DEV - (s + 1), N_DEV)
            y = gemm_block(bufR[s % DEPTH, :, :])
            local_max = jnp.maximum(local_max, jnp.max(y))
            out_ref[pl.ds(origin_r * m_per, m_per), :] = y
            if s < L_HOPS:
                origin_l = lax.rem(me + s + 1, N_DEV)
                y = gemm_block(bufL[s % DEPTH, :, :])
                local_max = jnp.maximum(local_max, jnp.max(y))
                out_ref[pl.ds(origin_l * m_per, m_per), :] = y

            rd_r[s].wait_send()
            if s < L_HOPS:
                rd_l[s].wait_send()
            if 1 <= s <= R_HOPS - DEPTH:
                pl.semaphore_signal(
                    creditR, inc=1, device_id=(left,),
                    device_id_type=pl.DeviceIdType.MESH)
            if 1 <= s <= L_HOPS - DEPTH:
                pl.semaphore_signal(
                    creditL, inc=1, device_id=(right,),
                    device_id_type=pl.DeviceIdType.MESH)

        amax_cur[...] = jnp.full((8, 128), local_max, jnp.float32)
        brd = []
        for d in range(1, N_DEV):
            tgt = lax.rem(me + d, N_DEV)
            rd = pltpu.make_async_remote_copy(
                src_ref=amax_cur, dst_ref=amax_all.at[d - 1],
                send_sem=bsend.at[d - 1], recv_sem=brecv.at[d - 1],
                device_id=(tgt,), device_id_type=pl.DeviceIdType.MESH,
            )
            rd.start()
            brd.append(rd)
        gmax = local_max
        for d in range(1, N_DEV):
            brd[d - 1].wait_recv()
            gmax = jnp.maximum(gmax, amax_all[d - 1, 0, 0])
        for d in range(1, N_DEV):
            brd[d - 1].wait_send()

        scale = gmax / 448.0
        q = (out_ref[...] / scale).astype(jnp.float8_e4m3fn)
        out_ref[...] = q.astype(jnp.float32) * scale

    return pl.pallas_call(
        body,
        out_shape=jax.ShapeDtypeStruct((N_DEV * m_per, n_per), jnp.float32),
        in_specs=[
            pl.BlockSpec(memory_space=pltpu.VMEM),
            pl.BlockSpec(memory_space=pltpu.VMEM),
        ],
        out_specs=pl.BlockSpec(memory_space=pltpu.VMEM),
        scratch_shapes=[
            pltpu.VMEM((DEPTH, m_per, k), jnp.float32),
            pltpu.VMEM((DEPTH, m_per, k), jnp.float32),
            pltpu.VMEM((8, 128), jnp.float32),
            pltpu.VMEM((N_PEERS, 8, 128), jnp.float32),
            pltpu.SemaphoreType.DMA((R_HOPS,)),
            pltpu.SemaphoreType.DMA((R_HOPS,)),
            pltpu.SemaphoreType.DMA((L_HOPS,)),
            pltpu.SemaphoreType.DMA((L_HOPS,)),
            pltpu.SemaphoreType.DMA((N_PEERS,)),
            pltpu.SemaphoreType.DMA((N_PEERS,)),
            pltpu.SemaphoreType.REGULAR,
            pltpu.SemaphoreType.REGULAR,
        ],
        compiler_params=pltpu.CompilerParams(
            vmem_limit_bytes=60 * 1024 * 1024,
        ),
    )(x, w_mat)
